# Initial kernel scaffold; baseline (speedup 1.0000x reference)
import functools

import jax
import jax.numpy as jnp
from jax import lax
from jax.experimental import pallas as pl
from jax.experimental.pallas import tpu as pltpu
from jax.experimental.pallas import tpu_sc as plsc

N = 10000
S = 10240
E = 160000
EP = 163840
NT = 16
NC = 2
B = 128
EPT = EP // NT
NBATCH = EPT // B
STRIP = S // NT
IN_F = 256
H = 512
C3 = 64
CW = 128
NCH = H // CW
CPC = NCH // NC
CW3 = C3 // NC
R = 256

_mesh = plsc.VectorSubcoreMesh(core_axis_name="c", subcore_axis_name="s")


@functools.partial(
    pl.kernel,
    out_type=jax.ShapeDtypeStruct((2 * S, 16), jnp.float32),
    mesh=_mesh,
    scratch_types=[
        pltpu.VMEM_SHARED((S, 16), jnp.float32),
        pltpu.VMEM((B,), jnp.int32),
        pltpu.VMEM((B, 16), jnp.float32),
    ],
)
def _deg_kernel(src_hbm, dst_hbm, ones_hbm, zeros_hbm, out_hbm,
                accum, idx_v, ones_v):
    c = lax.axis_index("c")
    s = lax.axis_index("s")
    pltpu.sync_copy(ones_hbm, ones_v)
    pltpu.sync_copy(zeros_hbm, accum.at[pl.ds(s * STRIP, STRIP)])
    plsc.subcore_barrier()

    def body(b, carry):
        base = s * EPT + b * B

        @pl.when(c == 0)
        def _():
            pltpu.sync_copy(src_hbm.at[pl.ds(base, B)], idx_v)

        @pl.when(c == 1)
        def _():
            pltpu.sync_copy(dst_hbm.at[pl.ds(base, B)], idx_v)

        pltpu.sync_copy(ones_v, accum.at[idx_v], add=True)
        return carry

    lax.fori_loop(0, NBATCH, body, 0)
    plsc.subcore_barrier()
    pltpu.sync_copy(accum.at[pl.ds(s * STRIP, STRIP)],
                    out_hbm.at[pl.ds(c * S + s * STRIP, STRIP)])


def _make_agg(cw, chunks_per_core):
    n_chunks = chunks_per_core * NC

    @functools.partial(
        pl.kernel,
        out_type=jax.ShapeDtypeStruct((n_chunks * S, cw), jnp.float32),
        mesh=_mesh,
        scratch_types=[
            pltpu.VMEM_SHARED((S, cw), jnp.float32),
            pltpu.VMEM((B,), jnp.int32),
            pltpu.VMEM((B,), jnp.int32),
            pltpu.VMEM((B, cw), jnp.float32),
            pltpu.VMEM((B, cw), jnp.float32),
            pltpu.SemaphoreType.DMA,
        ],
    )
    def _agg(tab_hbm, src_hbm, dst_hbm, zeros_hbm, out_hbm,
             accum, src_v, dst_v, rows_v, zeros_v, sem):
        c = lax.axis_index("c")
        s = lax.axis_index("s")
        pltpu.sync_copy(zeros_hbm, zeros_v)
        for ci in range(chunks_per_core):
            chunk = c * chunks_per_core + ci
            off = chunk * S
            for z in range(STRIP // B):
                pltpu.sync_copy(zeros_v, accum.at[pl.ds(s * STRIP + z * B, B)])
            plsc.subcore_barrier()

            def body(b, carry):
                base = s * EPT + b * B
                pltpu.sync_copy(src_hbm.at[pl.ds(base, B)], src_v)
                pltpu.sync_copy(dst_hbm.at[pl.ds(base, B)], dst_v)
                offv = jnp.full((16,), off, jnp.int32)
                for i in range(B // 16):
                    sl = pl.ds(i * 16, 16)
                    src_v[sl] = src_v[sl] + offv
                pltpu.async_copy(tab_hbm.at[src_v], rows_v, sem).wait()
                pltpu.sync_copy(rows_v, accum.at[dst_v], add=True)
                return carry

            lax.fori_loop(0, NBATCH, body, 0)
            plsc.subcore_barrier()
            pltpu.sync_copy(accum.at[pl.ds(s * STRIP, STRIP)],
                            out_hbm.at[pl.ds(off + s * STRIP, STRIP)])
            plsc.subcore_barrier()

    return _agg


_agg_h = _make_agg(CW, CPC)
_agg_o = _make_agg(CW3, 1)


def _tc1_body(x_ref, degs_ref, w_ref, y_ref):
    ns = lax.rsqrt(jnp.maximum(degs_ref[...], 1.0))
    acc = jnp.dot(x_ref[...] * ns, w_ref[...],
                  preferred_element_type=jnp.float32)
    for cc in range(NCH):
        y_ref[cc] = acc[:, cc * CW:(cc + 1) * CW]


def _tc1(x, deg_src, w1):
    return pl.pallas_call(
        _tc1_body,
        grid=(S // R,),
        in_specs=[
            pl.BlockSpec((R, IN_F), lambda i: (i, 0)),
            pl.BlockSpec((R, 1), lambda i: (i, 0)),
            pl.BlockSpec((IN_F, H), lambda i: (0, 0)),
        ],
        out_specs=pl.BlockSpec((NCH, R, CW), lambda i: (0, i, 0)),
        out_shape=jax.ShapeDtypeStruct((NCH, S, CW), jnp.float32),
    )(x, deg_src, w1)


def _make_tc_mid(out_w, out_chunks, out_cw):
    def body(agg_ref, degs_ref, degd_ref, b_ref, w_ref, h_ref, y_ref):
        ns = lax.rsqrt(jnp.maximum(degs_ref[...], 1.0))
        nd = lax.rsqrt(jnp.maximum(degd_ref[...], 1.0))
        acc = jnp.zeros((R, out_w), jnp.float32)
        for cc in range(NCH):
            t = jnp.maximum(agg_ref[cc] * nd + b_ref[0, cc * CW:(cc + 1) * CW],
                            0.0)
            h_ref[:, cc * CW:(cc + 1) * CW] = t
            acc = acc + jnp.dot(t * ns, w_ref[cc * CW:(cc + 1) * CW, :],
                                preferred_element_type=jnp.float32)
        for cc in range(out_chunks):
            y_ref[cc] = acc[:, cc * out_cw:(cc + 1) * out_cw]

    def call(agg, deg_src, deg_dst, bias, w):
        return pl.pallas_call(
            body,
            grid=(S // R,),
            in_specs=[
                pl.BlockSpec((NCH, R, CW), lambda i: (0, i, 0)),
                pl.BlockSpec((R, 1), lambda i: (i, 0)),
                pl.BlockSpec((R, 1), lambda i: (i, 0)),
                pl.BlockSpec((1, H), lambda i: (0, 0)),
                pl.BlockSpec((H, out_w), lambda i: (0, 0)),
            ],
            out_specs=[
                pl.BlockSpec((R, H), lambda i: (i, 0)),
                pl.BlockSpec((out_chunks, R, out_cw), lambda i: (0, i, 0)),
            ],
            out_shape=[
                jax.ShapeDtypeStruct((S, H), jnp.float32),
                jax.ShapeDtypeStruct((out_chunks, S, out_cw), jnp.float32),
            ],
        )(agg, deg_src, deg_dst, bias, w)

    return call


_tc2 = _make_tc_mid(H, NCH, CW)
_tc3 = _make_tc_mid(C3, NC, CW3)


def _tc4_body(agg_ref, degd_ref, b_ref, h_ref):
    nd = lax.rsqrt(jnp.maximum(degd_ref[...], 1.0))
    h = jnp.concatenate([agg_ref[0], agg_ref[1]], axis=1)
    h_ref[...] = h * nd + b_ref[...]


def _tc4(agg, deg_dst, bias):
    return pl.pallas_call(
        _tc4_body,
        grid=(S // R,),
        in_specs=[
            pl.BlockSpec((NC, R, CW3), lambda i: (0, i, 0)),
            pl.BlockSpec((R, 1), lambda i: (i, 0)),
            pl.BlockSpec((1, C3), lambda i: (0, 0)),
        ],
        out_specs=pl.BlockSpec((R, C3), lambda i: (i, 0)),
        out_shape=jax.ShapeDtypeStruct((S, C3), jnp.float32),
    )(agg, deg_dst, bias)


def kernel(features, edge_index, W1, b1, W2, b2, W3, b3):
    src = edge_index[0]
    dst = edge_index[1]
    pad = jnp.full((EP - E,), N, jnp.int32)
    src_pad = jnp.concatenate([src.astype(jnp.int32), pad])
    dst_pad = jnp.concatenate([dst.astype(jnp.int32), pad])
    x_pad = jnp.pad(features, ((0, S - N), (0, 0)))

    ones16 = jnp.ones((B, 16), jnp.float32)
    zeros16 = jnp.zeros((STRIP, 16), jnp.float32)
    zeros_cw = jnp.zeros((B, CW), jnp.float32)
    zeros_cw3 = jnp.zeros((B, CW3), jnp.float32)

    degs = _deg_kernel(src_pad, dst_pad, ones16, zeros16)
    deg_src = degs[:S, 0:1]
    deg_dst = degs[S:, 0:1]

    y1 = _tc1(x_pad, deg_src, W1)
    agg1 = _agg_h(y1.reshape(NCH * S, CW), src_pad, dst_pad, zeros_cw)
    h1, y2 = _tc2(agg1.reshape(NCH, S, CW), deg_src, deg_dst,
                  b1.reshape(1, H), W2)
    agg2 = _agg_h(y2.reshape(NCH * S, CW), src_pad, dst_pad, zeros_cw)
    h2, y3 = _tc3(agg2.reshape(NCH, S, CW), deg_src, deg_dst,
                  b2.reshape(1, H), W3)
    agg3 = _agg_o(y3.reshape(NC * S, CW3), src_pad, dst_pad, zeros_cw3)
    h3 = _tc4(agg3.reshape(NC, S, CW3), deg_dst, b3.reshape(1, C3))

    h1 = h1[:N]
    h2 = h2[:N]
    h3 = h3[:N]
    return (h3, (h1, h2, h3), h2)


# speedup vs baseline: 2.2442x; 2.2442x over previous
import functools

import jax
import jax.numpy as jnp
from jax import lax
from jax.experimental import pallas as pl
from jax.experimental.pallas import tpu as pltpu
from jax.experimental.pallas import tpu_sc as plsc

N = 10000
S = 10240
E = 160000
EP = 163840
NT = 16
NC = 2
B = 128
EPT = EP // NT
NBATCH = EPT // B
STRIP = S // NT
IN_F = 256
H = 512
C3 = 64
CW = 128
NCH = H // CW
CPC = NCH // NC
C3P = 128
R = 256
EPT3 = EP // (NC * NT)
NBATCH3 = EPT3 // B

_mesh = plsc.VectorSubcoreMesh(core_axis_name="c", subcore_axis_name="s")


@functools.partial(
    pl.kernel,
    out_type=jax.ShapeDtypeStruct((2 * S, CW), jnp.float32),
    mesh=_mesh,
    scratch_types=[
        pltpu.VMEM_SHARED((S, CW), jnp.float32),
        pltpu.VMEM((B,), jnp.int32),
        pltpu.VMEM((B, CW), jnp.float32),
        pltpu.VMEM((B, CW), jnp.float32),
    ],
)
def _deg_kernel(idx2_hbm, ones_hbm, zeros_hbm, out_hbm,
                accum, idx_v, ones_v, zeros_v):
    c = lax.axis_index("c")
    s = lax.axis_index("s")
    pltpu.sync_copy(ones_hbm, ones_v)
    pltpu.sync_copy(zeros_hbm, zeros_v)
    for z in range(STRIP // B):
        pltpu.sync_copy(zeros_v, accum.at[pl.ds(s * STRIP + z * B, B)])
    plsc.subcore_barrier()

    def body(b, carry):
        base = c * EP + s * EPT + b * B
        pltpu.sync_copy(idx2_hbm.at[pl.ds(base, B)], idx_v)
        pltpu.sync_copy(ones_v, accum.at[idx_v], add=True)
        return carry

    lax.fori_loop(0, NBATCH, body, 0)
    plsc.subcore_barrier()
    pltpu.sync_copy(accum.at[pl.ds(s * STRIP, STRIP)],
                    out_hbm.at[pl.ds(c * S + s * STRIP, STRIP)])


def _make_agg(cw, chunks_per_core):
    n_chunks = chunks_per_core * NC

    @functools.partial(
        pl.kernel,
        out_type=jax.ShapeDtypeStruct((n_chunks * S, cw), jnp.float32),
        mesh=_mesh,
        scratch_types=[
            pltpu.VMEM_SHARED((S, cw), jnp.float32),
            pltpu.VMEM((B,), jnp.int32),
            pltpu.VMEM((B,), jnp.int32),
            pltpu.VMEM((B, cw), jnp.float32),
            pltpu.VMEM((B, cw), jnp.float32),
            pltpu.SemaphoreType.DMA,
        ],
    )
    def _agg(tab_hbm, src_hbm, dst_hbm, zeros_hbm, out_hbm,
             accum, src_v, dst_v, rows_v, zeros_v, sem):
        c = lax.axis_index("c")
        s = lax.axis_index("s")
        pltpu.sync_copy(zeros_hbm, zeros_v)
        for ci in range(chunks_per_core):
            chunk = c * chunks_per_core + ci
            off = chunk * S
            for z in range(STRIP // B):
                pltpu.sync_copy(zeros_v, accum.at[pl.ds(s * STRIP + z * B, B)])
            plsc.subcore_barrier()

            def body(b, carry):
                base = s * EPT + b * B
                pltpu.sync_copy(src_hbm.at[pl.ds(base, B)], src_v)
                pltpu.sync_copy(dst_hbm.at[pl.ds(base, B)], dst_v)
                offv = jnp.full((16,), off, jnp.int32)
                for i in range(B // 16):
                    sl = pl.ds(i * 16, 16)
                    src_v[sl] = src_v[sl] + offv
                pltpu.async_copy(tab_hbm.at[src_v], rows_v, sem).wait()
                pltpu.sync_copy(rows_v, accum.at[dst_v], add=True)
                return carry

            lax.fori_loop(0, NBATCH, body, 0)
            plsc.subcore_barrier()
            pltpu.sync_copy(accum.at[pl.ds(s * STRIP, STRIP)],
                            out_hbm.at[pl.ds(off + s * STRIP, STRIP)])
            plsc.subcore_barrier()

    return _agg


_agg_h = _make_agg(CW, CPC)


@functools.partial(
    pl.kernel,
    out_type=jax.ShapeDtypeStruct((NC * S, C3P), jnp.float32),
    mesh=_mesh,
    scratch_types=[
        pltpu.VMEM_SHARED((S, C3P), jnp.float32),
        pltpu.VMEM((B,), jnp.int32),
        pltpu.VMEM((B,), jnp.int32),
        pltpu.VMEM((B, C3P), jnp.float32),
        pltpu.VMEM((B, C3P), jnp.float32),
        pltpu.SemaphoreType.DMA,
    ],
)
def _agg_o(tab_hbm, src_hbm, dst_hbm, zeros_hbm, out_hbm,
           accum, src_v, dst_v, rows_v, zeros_v, sem):
    c = lax.axis_index("c")
    s = lax.axis_index("s")
    pltpu.sync_copy(zeros_hbm, zeros_v)
    for z in range(STRIP // B):
        pltpu.sync_copy(zeros_v, accum.at[pl.ds(s * STRIP + z * B, B)])
    plsc.subcore_barrier()

    def body(b, carry):
        base = c * (EP // NC) + s * EPT3 + b * B
        pltpu.sync_copy(src_hbm.at[pl.ds(base, B)], src_v)
        pltpu.sync_copy(dst_hbm.at[pl.ds(base, B)], dst_v)
        pltpu.async_copy(tab_hbm.at[src_v], rows_v, sem).wait()
        pltpu.sync_copy(rows_v, accum.at[dst_v], add=True)
        return carry

    lax.fori_loop(0, NBATCH3, body, 0)
    plsc.subcore_barrier()
    pltpu.sync_copy(accum.at[pl.ds(s * STRIP, STRIP)],
                    out_hbm.at[pl.ds(c * S + s * STRIP, STRIP)])


def _tc1_body(x_ref, degs_ref, w_ref, y_ref):
    ns = lax.rsqrt(jnp.maximum(degs_ref[...], 1.0))
    acc = jnp.dot(x_ref[...] * ns, w_ref[...],
                  preferred_element_type=jnp.float32)
    for cc in range(NCH):
        y_ref[cc] = acc[:, cc * CW:(cc + 1) * CW]


def _tc1(x, deg_src, w1):
    return pl.pallas_call(
        _tc1_body,
        grid=(S // R,),
        in_specs=[
            pl.BlockSpec((R, IN_F), lambda i: (i, 0)),
            pl.BlockSpec((R, 1), lambda i: (i, 0)),
            pl.BlockSpec((IN_F, H), lambda i: (0, 0)),
        ],
        out_specs=pl.BlockSpec((NCH, R, CW), lambda i: (0, i, 0)),
        out_shape=jax.ShapeDtypeStruct((NCH, S, CW), jnp.float32),
    )(x, deg_src, w1)


def _make_tc_mid(out_w, out_chunks, out_cw):
    def body(agg_ref, degs_ref, degd_ref, b_ref, w_ref, h_ref, y_ref):
        ns = lax.rsqrt(jnp.maximum(degs_ref[...], 1.0))
        nd = lax.rsqrt(jnp.maximum(degd_ref[...], 1.0))
        acc = jnp.zeros((R, out_w), jnp.float32)
        for cc in range(NCH):
            t = jnp.maximum(agg_ref[cc] * nd + b_ref[0, cc * CW:(cc + 1) * CW],
                            0.0)
            h_ref[:, cc * CW:(cc + 1) * CW] = t
            acc = acc + jnp.dot(t * ns, w_ref[cc * CW:(cc + 1) * CW, :],
                                preferred_element_type=jnp.float32)
        for cc in range(out_chunks):
            y_ref[cc] = acc[:, cc * out_cw:(cc + 1) * out_cw]

    def call(agg, deg_src, deg_dst, bias, w):
        return pl.pallas_call(
            body,
            grid=(S // R,),
            in_specs=[
                pl.BlockSpec((NCH, R, CW), lambda i: (0, i, 0)),
                pl.BlockSpec((R, 1), lambda i: (i, 0)),
                pl.BlockSpec((R, 1), lambda i: (i, 0)),
                pl.BlockSpec((1, H), lambda i: (0, 0)),
                pl.BlockSpec((H, out_w), lambda i: (0, 0)),
            ],
            out_specs=[
                pl.BlockSpec((R, H), lambda i: (i, 0)),
                pl.BlockSpec((out_chunks, R, out_cw), lambda i: (0, i, 0)),
            ],
            out_shape=[
                jax.ShapeDtypeStruct((S, H), jnp.float32),
                jax.ShapeDtypeStruct((out_chunks, S, out_cw), jnp.float32),
            ],
        )(agg, deg_src, deg_dst, bias, w)

    return call


_tc2 = _make_tc_mid(H, NCH, CW)
_tc3 = _make_tc_mid(C3P, 1, C3P)


def _tc4_body(agg_ref, degd_ref, b_ref, h_ref):
    nd = lax.rsqrt(jnp.maximum(degd_ref[...], 1.0))
    h = (agg_ref[0] + agg_ref[1])[:, :C3]
    h_ref[...] = h * nd + b_ref[...]


def _tc4(agg, deg_dst, bias):
    return pl.pallas_call(
        _tc4_body,
        grid=(S // R,),
        in_specs=[
            pl.BlockSpec((NC, R, C3P), lambda i: (0, i, 0)),
            pl.BlockSpec((R, 1), lambda i: (i, 0)),
            pl.BlockSpec((1, C3), lambda i: (0, 0)),
        ],
        out_specs=pl.BlockSpec((R, C3), lambda i: (i, 0)),
        out_shape=jax.ShapeDtypeStruct((S, C3), jnp.float32),
    )(agg, deg_dst, bias)


def kernel(features, edge_index, W1, b1, W2, b2, W3, b3):
    src = edge_index[0]
    dst = edge_index[1]
    pad = jnp.full((EP - E,), N, jnp.int32)
    src_pad = jnp.concatenate([src.astype(jnp.int32), pad])
    dst_pad = jnp.concatenate([dst.astype(jnp.int32), pad])
    x_pad = jnp.pad(features, ((0, S - N), (0, 0)))

    ones_cw = jnp.ones((B, CW), jnp.float32)
    zeros_cw = jnp.zeros((B, CW), jnp.float32)
    w3_pad = jnp.pad(W3, ((0, 0), (0, C3P - C3)))

    idx2 = jnp.concatenate([src_pad, dst_pad])
    degs = _deg_kernel(idx2, ones_cw, zeros_cw)
    deg_src = degs[:S, 0:1]
    deg_dst = degs[S:, 0:1]

    y1 = _tc1(x_pad, deg_src, W1)
    agg1 = _agg_h(y1.reshape(NCH * S, CW), src_pad, dst_pad, zeros_cw)
    h1, y2 = _tc2(agg1.reshape(NCH, S, CW), deg_src, deg_dst,
                  b1.reshape(1, H), W2)
    agg2 = _agg_h(y2.reshape(NCH * S, CW), src_pad, dst_pad, zeros_cw)
    h2, y3 = _tc3(agg2.reshape(NCH, S, CW), deg_src, deg_dst,
                  b2.reshape(1, H), w3_pad)
    agg3 = _agg_o(y3.reshape(S, C3P), src_pad, dst_pad, zeros_cw)
    h3 = _tc4(agg3.reshape(NC, S, C3P), deg_dst, b3.reshape(1, C3))

    h1 = h1[:N]
    h2 = h2[:N]
    h3 = h3[:N]
    return (h3, (h1, h2, h3), h2)


# speedup vs baseline: 2.5576x; 1.1396x over previous
import functools

import jax
import jax.numpy as jnp
from jax import lax
from jax.experimental import pallas as pl
from jax.experimental.pallas import tpu as pltpu
from jax.experimental.pallas import tpu_sc as plsc

N = 10000
S = 10240
E = 160000
EP = 163840
NT = 16
NC = 2
B = 128
EPT = EP // NT
NBATCH = EPT // B
STRIP = S // NT
IN_F = 256
H = 512
C3 = 64
CW = 128
NCH = H // CW
CPC = NCH // NC
C3P = 128
R = 256
EPT3 = EP // (NC * NT)
NBATCH3 = EPT3 // B

_mesh = plsc.VectorSubcoreMesh(core_axis_name="c", subcore_axis_name="s")


@functools.partial(
    pl.kernel,
    out_type=jax.ShapeDtypeStruct((2 * S, CW), jnp.float32),
    mesh=_mesh,
    scratch_types=[
        pltpu.VMEM_SHARED((S, CW), jnp.float32),
        pltpu.VMEM((EP // B // NT, B), jnp.int32),
        pltpu.VMEM((B, CW), jnp.float32),
    ],
)
def _deg_kernel(idx2_hbm, ones_hbm, zeros_hbm, out_hbm,
                accum, idx_v, ones_v):
    c = lax.axis_index("c")
    s = lax.axis_index("s")
    nrows = EP // B // NT
    pltpu.sync_copy(ones_hbm, ones_v)
    pltpu.sync_copy(idx2_hbm.at[pl.ds(c * (EP // B) + s * nrows, nrows)],
                    idx_v)
    for z in range(STRIP // B):
        pltpu.sync_copy(zeros_hbm, accum.at[pl.ds(s * STRIP + z * B, B)])
    plsc.subcore_barrier()

    def body(b, carry):
        pltpu.sync_copy(ones_v, accum.at[idx_v.at[b]], add=True)
        return carry

    lax.fori_loop(0, nrows, body, 0)
    plsc.subcore_barrier()
    pltpu.sync_copy(accum.at[pl.ds(s * STRIP, STRIP)],
                    out_hbm.at[pl.ds(c * S + s * STRIP, STRIP)])


def _pipe(tab_hbm, sidx, didx, accum, rows, sem0, sem1, nbatch):
    pltpu.async_copy(tab_hbm.at[sidx.at[0]], rows.at[0], sem0)
    pltpu.async_copy(tab_hbm.at[sidx.at[1]], rows.at[1], sem1)

    def body(i, carry):
        b = i * 2
        pltpu.make_async_copy(tab_hbm.at[sidx.at[b]], rows.at[0], sem0).wait()
        pltpu.sync_copy(rows.at[0], accum.at[didx.at[b]], add=True)
        pltpu.async_copy(tab_hbm.at[sidx.at[b + 2]], rows.at[0], sem0)
        pltpu.make_async_copy(tab_hbm.at[sidx.at[b + 1]], rows.at[1],
                              sem1).wait()
        pltpu.sync_copy(rows.at[1], accum.at[didx.at[b + 1]], add=True)
        pltpu.async_copy(tab_hbm.at[sidx.at[b + 3]], rows.at[1], sem1)
        return carry

    lax.fori_loop(0, nbatch // 2 - 1, body, 0)
    b = nbatch - 2
    pltpu.make_async_copy(tab_hbm.at[sidx.at[b]], rows.at[0], sem0).wait()
    pltpu.sync_copy(rows.at[0], accum.at[didx.at[b]], add=True)
    pltpu.make_async_copy(tab_hbm.at[sidx.at[b + 1]], rows.at[1], sem1).wait()
    pltpu.sync_copy(rows.at[1], accum.at[didx.at[b + 1]], add=True)


EROWS = EP // B
GB = 64
GROWS = EP // GB
TROWS = GROWS // NT
HROWS = TROWS // 2
TROWS3 = GROWS // (NC * NT)


@functools.partial(
    pl.kernel,
    out_type=jax.ShapeDtypeStruct((NCH * S, CW), jnp.float32),
    mesh=_mesh,
    scratch_types=[
        pltpu.VMEM_SHARED((S, CW), jnp.float32),
        pltpu.VMEM((HROWS, GB), jnp.int32),
        pltpu.VMEM((HROWS, GB), jnp.int32),
        pltpu.VMEM((2, GB, CW), jnp.float32),
        pltpu.SemaphoreType.DMA,
        pltpu.SemaphoreType.DMA,
    ],
)
def _agg_h(tab_hbm, src4_hbm, dst2_hbm, zeros_hbm, out_hbm,
           accum, sidx, didx, rows, sem0, sem1):
    c = lax.axis_index("c")
    s = lax.axis_index("s")
    for ci in range(CPC):
        chunk = c * CPC + ci
        for z in range(STRIP // B):
            pltpu.sync_copy(zeros_hbm, accum.at[pl.ds(s * STRIP + z * B, B)])
        plsc.subcore_barrier()
        for half in range(2):
            pltpu.sync_copy(
                src4_hbm.at[pl.ds(chunk * GROWS + s * TROWS + half * HROWS,
                                  HROWS)], sidx)
            pltpu.sync_copy(dst2_hbm.at[pl.ds(s * TROWS + half * HROWS,
                                              HROWS)], didx)
            _pipe(tab_hbm, sidx, didx, accum, rows, sem0, sem1, HROWS)
        plsc.subcore_barrier()
        pltpu.sync_copy(accum.at[pl.ds(s * STRIP, STRIP)],
                        out_hbm.at[pl.ds(chunk * S + s * STRIP, STRIP)])
        plsc.subcore_barrier()


@functools.partial(
    pl.kernel,
    out_type=jax.ShapeDtypeStruct((NC * S, C3P), jnp.float32),
    mesh=_mesh,
    scratch_types=[
        pltpu.VMEM_SHARED((S, C3P), jnp.float32),
        pltpu.VMEM((TROWS3, GB), jnp.int32),
        pltpu.VMEM((TROWS3, GB), jnp.int32),
        pltpu.VMEM((2, GB, C3P), jnp.float32),
        pltpu.SemaphoreType.DMA,
        pltpu.SemaphoreType.DMA,
    ],
)
def _agg_o(tab_hbm, src2_hbm, dst2_hbm, zeros_hbm, out_hbm,
           accum, sidx, didx, rows, sem0, sem1):
    c = lax.axis_index("c")
    s = lax.axis_index("s")
    base = c * (GROWS // NC) + s * TROWS3
    pltpu.sync_copy(src2_hbm.at[pl.ds(base, TROWS3)], sidx)
    pltpu.sync_copy(dst2_hbm.at[pl.ds(base, TROWS3)], didx)
    for z in range(STRIP // B):
        pltpu.sync_copy(zeros_hbm, accum.at[pl.ds(s * STRIP + z * B, B)])
    plsc.subcore_barrier()
    _pipe(tab_hbm, sidx, didx, accum, rows, sem0, sem1, TROWS3)
    plsc.subcore_barrier()
    pltpu.sync_copy(accum.at[pl.ds(s * STRIP, STRIP)],
                    out_hbm.at[pl.ds(c * S + s * STRIP, STRIP)])


def _tc1_body(x_ref, degs_ref, w_ref, y_ref):
    ns = lax.rsqrt(jnp.maximum(degs_ref[...], 1.0))
    acc = jnp.dot(x_ref[...] * ns, w_ref[...],
                  preferred_element_type=jnp.float32)
    for cc in range(NCH):
        y_ref[cc] = acc[:, cc * CW:(cc + 1) * CW]


def _tc1(x, deg_src, w1):
    return pl.pallas_call(
        _tc1_body,
        grid=(S // R,),
        in_specs=[
            pl.BlockSpec((R, IN_F), lambda i: (i, 0)),
            pl.BlockSpec((R, 1), lambda i: (i, 0)),
            pl.BlockSpec((IN_F, H), lambda i: (0, 0)),
        ],
        out_specs=pl.BlockSpec((NCH, R, CW), lambda i: (0, i, 0)),
        out_shape=jax.ShapeDtypeStruct((NCH, S, CW), jnp.float32),
    )(x, deg_src, w1)


def _make_tc_mid(out_w, out_chunks, out_cw):
    def body(agg_ref, degs_ref, degd_ref, b_ref, w_ref, h_ref, y_ref):
        ns = lax.rsqrt(jnp.maximum(degs_ref[...], 1.0))
        nd = lax.rsqrt(jnp.maximum(degd_ref[...], 1.0))
        acc = jnp.zeros((R, out_w), jnp.float32)
        for cc in range(NCH):
            t = jnp.maximum(agg_ref[cc] * nd + b_ref[0, cc * CW:(cc + 1) * CW],
                            0.0)
            h_ref[:, cc * CW:(cc + 1) * CW] = t
            acc = acc + jnp.dot(t * ns, w_ref[cc * CW:(cc + 1) * CW, :],
                                preferred_element_type=jnp.float32)
        for cc in range(out_chunks):
            y_ref[cc] = acc[:, cc * out_cw:(cc + 1) * out_cw]

    def call(agg, deg_src, deg_dst, bias, w):
        return pl.pallas_call(
            body,
            grid=(S // R,),
            in_specs=[
                pl.BlockSpec((NCH, R, CW), lambda i: (0, i, 0)),
                pl.BlockSpec((R, 1), lambda i: (i, 0)),
                pl.BlockSpec((R, 1), lambda i: (i, 0)),
                pl.BlockSpec((1, H), lambda i: (0, 0)),
                pl.BlockSpec((H, out_w), lambda i: (0, 0)),
            ],
            out_specs=[
                pl.BlockSpec((R, H), lambda i: (i, 0)),
                pl.BlockSpec((out_chunks, R, out_cw), lambda i: (0, i, 0)),
            ],
            out_shape=[
                jax.ShapeDtypeStruct((S, H), jnp.float32),
                jax.ShapeDtypeStruct((out_chunks, S, out_cw), jnp.float32),
            ],
        )(agg, deg_src, deg_dst, bias, w)

    return call


_tc2 = _make_tc_mid(H, NCH, CW)
_tc3 = _make_tc_mid(C3P, 1, C3P)


def _tc4_body(agg_ref, degd_ref, b_ref, h_ref):
    nd = lax.rsqrt(jnp.maximum(degd_ref[...], 1.0))
    h = (agg_ref[0] + agg_ref[1])[:, :C3]
    h_ref[...] = h * nd + b_ref[...]


def _tc4(agg, deg_dst, bias):
    return pl.pallas_call(
        _tc4_body,
        grid=(S // R,),
        in_specs=[
            pl.BlockSpec((NC, R, C3P), lambda i: (0, i, 0)),
            pl.BlockSpec((R, 1), lambda i: (i, 0)),
            pl.BlockSpec((1, C3), lambda i: (0, 0)),
        ],
        out_specs=pl.BlockSpec((R, C3), lambda i: (i, 0)),
        out_shape=jax.ShapeDtypeStruct((S, C3), jnp.float32),
    )(agg, deg_dst, bias)


def kernel(features, edge_index, W1, b1, W2, b2, W3, b3):
    src = edge_index[0]
    dst = edge_index[1]
    pad = jnp.full((EP - E,), N, jnp.int32)
    src_pad = jnp.concatenate([src.astype(jnp.int32), pad])
    dst_pad = jnp.concatenate([dst.astype(jnp.int32), pad])
    x_pad = jnp.pad(features, ((0, S - N), (0, 0)))

    ones_cw = jnp.ones((B, CW), jnp.float32)
    zeros_cw = jnp.zeros((B, CW), jnp.float32)
    w3_pad = jnp.pad(W3, ((0, 0), (0, C3P - C3)))

    idx2 = jnp.concatenate([src_pad, dst_pad]).reshape(2 * EROWS, B)
    src2 = src_pad.reshape(GROWS, GB)
    dst2 = dst_pad.reshape(GROWS, GB)
    offs = (jnp.arange(NCH, dtype=jnp.int32) * S)[:, None]
    src4 = (src_pad[None, :] + offs).reshape(NCH * GROWS, GB)

    degs = _deg_kernel(idx2, ones_cw, zeros_cw)
    deg_src = degs[:S, 0:1]
    deg_dst = degs[S:, 0:1]

    y1 = _tc1(x_pad, deg_src, W1)
    agg1 = _agg_h(y1.reshape(NCH * S, CW), src4, dst2, zeros_cw)
    h1, y2 = _tc2(agg1.reshape(NCH, S, CW), deg_src, deg_dst,
                  b1.reshape(1, H), W2)
    agg2 = _agg_h(y2.reshape(NCH * S, CW), src4, dst2, zeros_cw)
    h2, y3 = _tc3(agg2.reshape(NCH, S, CW), deg_src, deg_dst,
                  b2.reshape(1, H), w3_pad)
    agg3 = _agg_o(y3.reshape(S, C3P), src2, dst2, zeros_cw)
    h3 = _tc4(agg3.reshape(NC, S, C3P), deg_dst, b3.reshape(1, C3))

    h1 = h1[:N]
    h2 = h2[:N]
    h3 = h3[:N]
    return (h3, (h1, h2, h3), h2)
